# baseline (speedup 1.0000x reference)
import functools

import jax
import jax.numpy as jnp
from jax import lax
from jax.experimental import pallas as pl
from jax.experimental.pallas import tpu as pltpu
from jax.experimental.pallas import tpu_sc as plsc

D = 128
ROWS = 4096
COLS = 200
B = ROWS * COLS

_info = plsc.get_sparse_core_info()
NC = _info.num_cores
NS = _info.num_subcores
NW = NC * NS
PER_W = B // NW

RPG = 128
K = 2
CHUNK = K * RPG
NCHUNK = PER_W // CHUNK
GROUPS = PER_W // RPG

_mesh = plsc.VectorSubcoreMesh(core_axis_name="c", subcore_axis_name="s")


@functools.partial(
    pl.kernel,
    mesh=_mesh,
    out_type=jax.ShapeDtypeStruct((B, D), jnp.float32),
    scratch_types=[
        pltpu.VMEM((PER_W,), jnp.int32),
        pltpu.VMEM((3, CHUNK, D), jnp.float32),
        pltpu.SemaphoreType.DMA,
        pltpu.SemaphoreType.DMA,
        pltpu.SemaphoreType.DMA,
        pltpu.SemaphoreType.DMA,
        pltpu.SemaphoreType.DMA,
        pltpu.SemaphoreType.DMA,
    ],
)
def _emb_lookup(ids_hbm, table_hbm, out_hbm, idx_v, rows_v, g0, g1, g2, o0, o1, o2):
    wid = lax.axis_index("c") * NS + lax.axis_index("s")
    base = wid * PER_W
    gsem = (g0, g1, g2)
    osem = (o0, o1, o2)

    pltpu.sync_copy(ids_hbm.at[pl.ds(base, PER_W)], idx_v)

    def fire_gather(i, s):
        for j in range(K):
            pltpu.async_copy(
                table_hbm.at[idx_v.at[pl.ds((i * K + j) * RPG, RPG)]],
                rows_v.at[s, pl.ds(j * RPG, RPG)],
                gsem[s],
            )

    def drain_gather(s):
        pltpu.make_async_copy(
            table_hbm.at[pl.ds(0, CHUNK)], rows_v.at[s], gsem[s]
        ).wait()

    def fire_out(i, s):
        pltpu.async_copy(
            rows_v.at[s], out_hbm.at[pl.ds(base + i * CHUNK, CHUNK)], osem[s]
        )

    def drain_out(s):
        pltpu.make_async_copy(
            rows_v.at[s], out_hbm.at[pl.ds(0, CHUNK)], osem[s]
        ).wait()

    def step(i, s):
        drain_out((s + 2) % 3)
        fire_gather(i + 2, (s + 2) % 3)
        drain_gather(s)
        fire_out(i, s)

    fire_gather(0, 0)
    fire_gather(1, 1)
    fire_gather(2, 2)
    drain_gather(0)
    fire_out(0, 0)
    step(1, 1)

    def body(u, carry):
        i = 3 * u + 2
        step(i, 2)
        step(i + 1, 0)
        step(i + 2, 1)
        return carry

    lax.fori_loop(0, (NCHUNK - 4) // 3, body, 0)

    drain_out(1)
    drain_gather(2)
    fire_out(NCHUNK - 2, 2)
    drain_out(2)
    drain_gather(0)
    fire_out(NCHUNK - 1, 0)
    drain_out(0)


def kernel(input_ids, table):
    ids = input_ids.reshape(B).astype(jnp.int32)
    out = _emb_lookup(ids, table)
    return out.reshape(ROWS, COLS, D)


# speedup vs baseline: 1.0014x; 1.0014x over previous
import functools

import jax
import jax.numpy as jnp
from jax import lax
from jax.experimental import pallas as pl
from jax.experimental.pallas import tpu as pltpu
from jax.experimental.pallas import tpu_sc as plsc

D = 128
ROWS = 4096
COLS = 200
B = ROWS * COLS

_info = plsc.get_sparse_core_info()
NC = _info.num_cores
NS = _info.num_subcores
NW = NC * NS
PER_W = B // NW

RPG = 128
K = 2
CHUNK = K * RPG
NCHUNK = PER_W // CHUNK
GROUPS = PER_W // RPG

_mesh = plsc.VectorSubcoreMesh(core_axis_name="c", subcore_axis_name="s")


@functools.partial(
    pl.kernel,
    mesh=_mesh,
    out_type=jax.ShapeDtypeStruct((B, D), jnp.float32),
    scratch_types=[
        pltpu.VMEM((GROUPS, RPG), jnp.int32),
        pltpu.VMEM((3, CHUNK, D), jnp.float32),
        pltpu.SemaphoreType.DMA,
        pltpu.SemaphoreType.DMA,
        pltpu.SemaphoreType.DMA,
        pltpu.SemaphoreType.DMA,
        pltpu.SemaphoreType.DMA,
        pltpu.SemaphoreType.DMA,
    ],
)
def _emb_lookup(ids_hbm, table_hbm, out_hbm, idx_v, rows_v, g0, g1, g2, o0, o1, o2):
    wid = lax.axis_index("s") * NC + lax.axis_index("c")
    base = wid * PER_W
    gsem = (g0, g1, g2)
    osem = (o0, o1, o2)

    pltpu.sync_copy(ids_hbm.at[wid], idx_v)

    def fire_gather(i, s):
        for j in range(K):
            pltpu.async_copy(
                table_hbm.at[idx_v.at[i * K + j]],
                rows_v.at[s, pl.ds(j * RPG, RPG)],
                gsem[s],
            )

    def drain_gather(s):
        pltpu.make_async_copy(
            table_hbm.at[pl.ds(0, CHUNK)], rows_v.at[s], gsem[s]
        ).wait()

    def fire_out(i, s):
        pltpu.async_copy(
            rows_v.at[s], out_hbm.at[pl.ds(base + i * CHUNK, CHUNK)], osem[s]
        )

    def drain_out(s):
        pltpu.make_async_copy(
            rows_v.at[s], out_hbm.at[pl.ds(0, CHUNK)], osem[s]
        ).wait()

    def step(i, s):
        drain_out((s + 2) % 3)
        fire_gather(i + 2, (s + 2) % 3)
        drain_gather(s)
        fire_out(i, s)

    fire_gather(0, 0)
    fire_gather(1, 1)
    fire_gather(2, 2)
    drain_gather(0)
    fire_out(0, 0)
    step(1, 1)

    def body(u, carry):
        i = 3 * u + 2
        step(i, 2)
        step(i + 1, 0)
        step(i + 2, 1)
        return carry

    lax.fori_loop(0, (NCHUNK - 4) // 3, body, 0)

    drain_out(1)
    drain_gather(2)
    fire_out(NCHUNK - 2, 2)
    drain_out(2)
    drain_gather(0)
    fire_out(NCHUNK - 1, 0)
    drain_out(0)


def kernel(input_ids, table):
    ids = input_ids.reshape(NW, GROUPS, RPG).astype(jnp.int32)
    out = _emb_lookup(ids, table)
    return out.reshape(ROWS, COLS, D)


# speedup vs baseline: 1.0154x; 1.0139x over previous
import functools

import jax
import jax.numpy as jnp
from jax import lax
from jax.experimental import pallas as pl
from jax.experimental.pallas import tpu as pltpu
from jax.experimental.pallas import tpu_sc as plsc

D = 128
ROWS = 4096
COLS = 200
B = ROWS * COLS

_info = plsc.get_sparse_core_info()
NC = _info.num_cores
NS = _info.num_subcores
NW = NC * NS
RPW = ROWS // NW
PER_W = RPW * COLS

CHUNK = COLS
NCHUNK = RPW
G1 = 128
G2 = COLS - G1

_mesh = plsc.VectorSubcoreMesh(core_axis_name="c", subcore_axis_name="s")


@functools.partial(
    pl.kernel,
    mesh=_mesh,
    out_type=jax.ShapeDtypeStruct((B, D), jnp.float32),
    scratch_types=[
        pltpu.VMEM((RPW, COLS), jnp.int32),
        pltpu.VMEM((3, CHUNK, D), jnp.float32),
        pltpu.SemaphoreType.DMA,
        pltpu.SemaphoreType.DMA,
        pltpu.SemaphoreType.DMA,
        pltpu.SemaphoreType.DMA,
        pltpu.SemaphoreType.DMA,
        pltpu.SemaphoreType.DMA,
    ],
)
def _emb_lookup(ids_hbm, table_hbm, out_hbm, idx_v, rows_v, g0, g1, g2, o0, o1, o2):
    wid = lax.axis_index("c") * NS + lax.axis_index("s")
    base = wid * PER_W
    gsem = (g0, g1, g2)
    osem = (o0, o1, o2)

    pltpu.sync_copy(ids_hbm.at[pl.ds(wid * RPW, RPW)], idx_v)

    def fire_gather(i, s):
        pltpu.async_copy(
            table_hbm.at[idx_v.at[i, pl.ds(0, G1)]],
            rows_v.at[s, pl.ds(0, G1)],
            gsem[s],
        )
        pltpu.async_copy(
            table_hbm.at[idx_v.at[i, pl.ds(G1, G2)]],
            rows_v.at[s, pl.ds(G1, G2)],
            gsem[s],
        )

    def drain_gather(s):
        pltpu.make_async_copy(
            table_hbm.at[pl.ds(0, CHUNK)], rows_v.at[s], gsem[s]
        ).wait()

    def fire_out(i, s):
        pltpu.async_copy(
            rows_v.at[s], out_hbm.at[pl.ds(base + i * CHUNK, CHUNK)], osem[s]
        )

    def drain_out(s):
        pltpu.make_async_copy(
            rows_v.at[s], out_hbm.at[pl.ds(0, CHUNK)], osem[s]
        ).wait()

    def step(i, s):
        drain_out((s + 2) % 3)
        fire_gather(i + 2, (s + 2) % 3)
        drain_gather(s)
        fire_out(i, s)

    fire_gather(0, 0)
    fire_gather(1, 1)
    fire_gather(2, 2)
    drain_gather(0)
    fire_out(0, 0)
    step(1, 1)

    def body(u, carry):
        i = 3 * u + 2
        step(i, 2)
        step(i + 1, 0)
        step(i + 2, 1)
        return carry

    lax.fori_loop(0, (NCHUNK - 8) // 3, body, 0)
    step(NCHUNK - 6, 2)
    step(NCHUNK - 5, 0)
    step(NCHUNK - 4, 1)
    step(NCHUNK - 3, 2)

    drain_out(2)
    drain_gather(0)
    fire_out(NCHUNK - 2, 0)
    drain_out(0)
    drain_gather(1)
    fire_out(NCHUNK - 1, 1)
    drain_out(1)


def kernel(input_ids, table):
    out = _emb_lookup(input_ids.astype(jnp.int32), table)
    return out.reshape(ROWS, COLS, D)
